# Initial kernel scaffold; baseline (speedup 1.0000x reference)
import functools

import jax
import jax.numpy as jnp
from jax import lax
from jax.experimental import pallas as pl
from jax.experimental.pallas import tpu as pltpu
from jax.experimental.pallas import tpu_sc as plsc

N = 10000
E = 320000
D = 128

NC = 2
NS = 16
NW = NC * NS
CHUNK = 128
CH = 79
EPAD = NW * CH * CHUNK
NPAD = 10240
DUMMY = N
ROWS_PER_TILE = NPAD // NS

_MESH = plsc.VectorSubcoreMesh(core_axis_name="c", subcore_axis_name="s")


def _sc_degree(dst_w, ones_hbm, zeros16_hbm):
    @functools.partial(
        pl.kernel,
        out_type=jax.ShapeDtypeStruct((NC, NPAD, 16), jnp.float32),
        mesh=_MESH,
        scratch_types=[
            pltpu.VMEM((CH, CHUNK), jnp.int32),
            pltpu.VMEM((CHUNK, 16), jnp.float32),
        ],
    )
    def k(dst_hbm, ones_h, zeros_h, out_hbm, dst_v, ones_v):
        c = lax.axis_index("c")
        s = lax.axis_index("s")
        w = s * NC + c
        stripe = s * ROWS_PER_TILE

        pltpu.sync_copy(dst_hbm.at[w], dst_v)
        pltpu.sync_copy(ones_h, ones_v)

        def run(dacc):
            for b in range(ROWS_PER_TILE // CHUNK):
                pltpu.sync_copy(zeros_h,
                                dacc.at[pl.ds(stripe + b * CHUNK, CHUNK)])
            plsc.subcore_barrier()

            def body(j, carry):
                pltpu.sync_copy(ones_v, dacc.at[dst_v.at[j]], add=True)
                return carry

            lax.fori_loop(0, CH, body, 0)
            plsc.subcore_barrier()
            pltpu.sync_copy(dacc.at[pl.ds(stripe, ROWS_PER_TILE)],
                            out_hbm.at[c, pl.ds(stripe, ROWS_PER_TILE)])

        pl.run_scoped(run, pltpu.VMEM_SHARED((NPAD, 16), jnp.float32))

    return k(dst_w, ones_hbm, zeros16_hbm)


def _sc_scatter(g, src_w, dst_w, zeros_hbm):
    @functools.partial(
        pl.kernel,
        out_type=jax.ShapeDtypeStruct((NC, NPAD, D), jnp.float32),
        mesh=_MESH,
        scratch_types=[
            pltpu.VMEM((CH, CHUNK), jnp.int32),
            pltpu.VMEM((CH, CHUNK), jnp.int32),
            pltpu.VMEM((CHUNK, D), jnp.float32),
            pltpu.VMEM((CHUNK, D), jnp.float32),
            pltpu.SemaphoreType.DMA,
            pltpu.SemaphoreType.DMA,
        ],
    )
    def k(g_hbm, src_hbm, dst_hbm, zeros_h, out_hbm,
          src_v, dst_v, buf0, buf1, sem0, sem1):
        c = lax.axis_index("c")
        s = lax.axis_index("s")
        w = s * NC + c
        stripe = s * ROWS_PER_TILE

        pltpu.sync_copy(src_hbm.at[w], src_v)
        pltpu.sync_copy(dst_hbm.at[w], dst_v)

        def run(acc):
            for b in range(ROWS_PER_TILE // CHUNK):
                pltpu.sync_copy(zeros_h,
                                acc.at[pl.ds(stripe + b * CHUNK, CHUNK)])
            plsc.subcore_barrier()

            pltpu.async_copy(g_hbm.at[src_v.at[0]], buf0, sem0)

            def body(p, carry):
                j = 2 * p
                pltpu.async_copy(g_hbm.at[src_v.at[j + 1]], buf1, sem1)
                pltpu.make_async_copy(g_hbm.at[src_v.at[j]], buf0, sem0).wait()
                pltpu.sync_copy(buf0, acc.at[dst_v.at[j]], add=True)
                pltpu.async_copy(g_hbm.at[src_v.at[j + 2]], buf0, sem0)
                pltpu.make_async_copy(g_hbm.at[src_v.at[j + 1]], buf1,
                                      sem1).wait()
                pltpu.sync_copy(buf1, acc.at[dst_v.at[j + 1]], add=True)
                return carry

            lax.fori_loop(0, (CH - 1) // 2, body, 0)
            pltpu.make_async_copy(g_hbm.at[src_v.at[CH - 1]], buf0, sem0).wait()
            pltpu.sync_copy(buf0, acc.at[dst_v.at[CH - 1]], add=True)

            plsc.subcore_barrier()
            pltpu.sync_copy(acc.at[pl.ds(stripe, ROWS_PER_TILE)],
                            out_hbm.at[c, pl.ds(stripe, ROWS_PER_TILE)])

        pl.run_scoped(run, pltpu.VMEM_SHARED((NPAD, D), jnp.float32))

    return k(g, src_w, dst_w, zeros_hbm)


def _tc_dis(dega):
    def body(dega_ref, dis_ref):
        deg = dega_ref[0, :, 0:1] + dega_ref[1, :, 0:1] + 1.0
        dis_ref[...] = lax.rsqrt(deg)

    return pl.pallas_call(
        body,
        out_shape=jax.ShapeDtypeStruct((NPAD, 1), jnp.float32),
    )(dega)


_BLK = 1280
_GRID = NPAD // _BLK


def _tc_matmul_scale(x, w, dis):
    def body(x_ref, w_ref, dis_ref, g_ref):
        h = jnp.dot(x_ref[...], w_ref[...], preferred_element_type=jnp.float32)
        g_ref[...] = dis_ref[...] * h

    return pl.pallas_call(
        body,
        grid=(_GRID,),
        in_specs=[
            pl.BlockSpec((_BLK, D), lambda i: (i, 0)),
            pl.BlockSpec((D, D), lambda i: (0, 0)),
            pl.BlockSpec((_BLK, 1), lambda i: (i, 0)),
        ],
        out_specs=pl.BlockSpec((_BLK, D), lambda i: (i, 0)),
        out_shape=jax.ShapeDtypeStruct((NPAD, D), jnp.float32),
    )(x, w, dis)


def _tc_combine_matmul(acc, g_prev, dis, w):
    def body(acc_ref, g_ref, dis_ref, w_ref, out_ref):
        pre = dis_ref[...] * (acc_ref[0] + acc_ref[1] + g_ref[...])
        x2 = jnp.maximum(pre, 0.0)
        h = jnp.dot(x2, w_ref[...], preferred_element_type=jnp.float32)
        out_ref[...] = dis_ref[...] * h

    return pl.pallas_call(
        body,
        grid=(_GRID,),
        in_specs=[
            pl.BlockSpec((NC, _BLK, D), lambda i: (0, i, 0)),
            pl.BlockSpec((_BLK, D), lambda i: (i, 0)),
            pl.BlockSpec((_BLK, 1), lambda i: (i, 0)),
            pl.BlockSpec((D, D), lambda i: (0, 0)),
        ],
        out_specs=pl.BlockSpec((_BLK, D), lambda i: (i, 0)),
        out_shape=jax.ShapeDtypeStruct((NPAD, D), jnp.float32),
    )(acc, g_prev, dis, w)


def _tc_final(acc, g_prev, dis):
    def body(acc_ref, g_ref, dis_ref, out_ref):
        out_ref[...] = dis_ref[...] * (acc_ref[0] + acc_ref[1] + g_ref[...])

    return pl.pallas_call(
        body,
        grid=(_GRID,),
        in_specs=[
            pl.BlockSpec((NC, _BLK, D), lambda i: (0, i, 0)),
            pl.BlockSpec((_BLK, D), lambda i: (i, 0)),
            pl.BlockSpec((_BLK, 1), lambda i: (i, 0)),
        ],
        out_specs=pl.BlockSpec((_BLK, D), lambda i: (i, 0)),
        out_shape=jax.ShapeDtypeStruct((NPAD, D), jnp.float32),
    )(acc, g_prev, dis)


def kernel(x, edge_index, W0, W1, W2):
    x_p = jnp.zeros((NPAD, D), jnp.float32).at[:N].set(x)
    src = jnp.full((EPAD,), DUMMY, jnp.int32).at[:E].set(edge_index[0])
    dst = jnp.full((EPAD,), DUMMY, jnp.int32).at[:E].set(edge_index[1])
    src_w = src.reshape(NW, CH, CHUNK)
    dst_w = dst.reshape(NW, CH, CHUNK)
    ones16 = jnp.ones((CHUNK, 16), jnp.float32)
    zeros16 = jnp.zeros((CHUNK, 16), jnp.float32)
    zerosD = jnp.zeros((CHUNK, D), jnp.float32)

    dega = _sc_degree(dst_w, ones16, zeros16)
    dis = _tc_dis(dega)

    g0 = _tc_matmul_scale(x_p, W0, dis)
    acc0 = _sc_scatter(g0, src_w, dst_w, zerosD)
    g1 = _tc_combine_matmul(acc0, g0, dis, W1)
    acc1 = _sc_scatter(g1, src_w, dst_w, zerosD)
    g2 = _tc_combine_matmul(acc1, g1, dis, W2)
    acc2 = _sc_scatter(g2, src_w, dst_w, zerosD)

    out_p = _tc_final(acc2, g2, dis)
    return out_p[:N]


# speedup vs baseline: 7.5751x; 7.5751x over previous
import functools

import jax
import jax.numpy as jnp
from jax import lax
from jax.experimental import pallas as pl
from jax.experimental.pallas import tpu as pltpu
from jax.experimental.pallas import tpu_sc as plsc

N = 10000
E = 320000
D = 128

NC = 2
NS = 16
NW = NC * NS
CHUNK = 128
CH = 80
PIECE = 16
NPIECE = CH // PIECE
EPAD = NW * CH * CHUNK
NPAD = 10240
DUMMY = N
ROWS_PER_TILE = NPAD // NS

_MESH = plsc.VectorSubcoreMesh(core_axis_name="c", subcore_axis_name="s")


def _sc_degree(dst_w, ones_hbm, zeros16_hbm):
    @functools.partial(
        pl.kernel,
        out_type=jax.ShapeDtypeStruct((NC, NPAD, D), jnp.float32),
        mesh=_MESH,
        scratch_types=[
            pltpu.VMEM((CH, CHUNK), jnp.int32),
            pltpu.VMEM((CHUNK, D), jnp.float32),
            pltpu.VMEM_SHARED((NPAD, D), jnp.float32),
        ],
    )
    def k(dst_hbm, ones_h, zeros_h, out_hbm, dst_v, ones_v, dacc):
        c = lax.axis_index("c")
        s = lax.axis_index("s")
        w = s * NC + c
        stripe = s * ROWS_PER_TILE

        pltpu.sync_copy(dst_hbm.at[w], dst_v)
        pltpu.sync_copy(ones_h, ones_v)

        for b in range(ROWS_PER_TILE // CHUNK):
            pltpu.sync_copy(zeros_h,
                            dacc.at[pl.ds(stripe + b * CHUNK, CHUNK)])
        plsc.subcore_barrier()

        def body(j, carry):
            pltpu.sync_copy(ones_v, dacc.at[dst_v.at[j]], add=True)
            return carry

        lax.fori_loop(0, CH, body, 0)
        plsc.subcore_barrier()
        pltpu.sync_copy(dacc.at[pl.ds(stripe, ROWS_PER_TILE)],
                        out_hbm.at[c, pl.ds(stripe, ROWS_PER_TILE)])

    return k(dst_w, ones_hbm, zeros16_hbm)


def _sc_scatter(g, src_w, dst_w, zeros_hbm):
    @functools.partial(
        pl.kernel,
        out_type=jax.ShapeDtypeStruct((NC, NPAD, D), jnp.float32),
        mesh=_MESH,
        scratch_types=[
            pltpu.VMEM((PIECE, CHUNK), jnp.int32),
            pltpu.VMEM((PIECE, CHUNK), jnp.int32),
            pltpu.VMEM((CHUNK, D), jnp.float32),
            pltpu.VMEM((CHUNK, D), jnp.float32),
            pltpu.SemaphoreType.DMA,
            pltpu.SemaphoreType.DMA,
            pltpu.VMEM_SHARED((NPAD, D), jnp.float32),
        ],
    )
    def k(g_hbm, src_hbm, dst_hbm, zeros_h, out_hbm,
          src_v, dst_v, buf0, buf1, sem0, sem1, acc):
        c = lax.axis_index("c")
        s = lax.axis_index("s")
        w = s * NC + c
        stripe = s * ROWS_PER_TILE

        for b in range(ROWS_PER_TILE // CHUNK):
            pltpu.sync_copy(zeros_h,
                            acc.at[pl.ds(stripe + b * CHUNK, CHUNK)])
        plsc.subcore_barrier()

        def piece(q, carry):
            pltpu.sync_copy(src_hbm.at[w, pl.ds(q * PIECE, PIECE)], src_v)
            pltpu.sync_copy(dst_hbm.at[w, pl.ds(q * PIECE, PIECE)], dst_v)
            pltpu.async_copy(g_hbm.at[src_v.at[0]], buf0, sem0)

            def pair(p, c2):
                j = 2 * p
                pltpu.async_copy(g_hbm.at[src_v.at[j + 1]], buf1, sem1)
                pltpu.make_async_copy(g_hbm.at[src_v.at[j]], buf0, sem0).wait()
                pltpu.sync_copy(buf0, acc.at[dst_v.at[j]], add=True)

                @pl.when(j + 2 < PIECE)
                def _():
                    pltpu.async_copy(g_hbm.at[src_v.at[j + 2]], buf0, sem0)

                pltpu.make_async_copy(g_hbm.at[src_v.at[j + 1]], buf1,
                                      sem1).wait()
                pltpu.sync_copy(buf1, acc.at[dst_v.at[j + 1]], add=True)
                return c2

            lax.fori_loop(0, PIECE // 2, pair, 0)
            return carry

        lax.fori_loop(0, NPIECE, piece, 0)

        plsc.subcore_barrier()
        pltpu.sync_copy(acc.at[pl.ds(stripe, ROWS_PER_TILE)],
                        out_hbm.at[c, pl.ds(stripe, ROWS_PER_TILE)])

    return k(g, src_w, dst_w, zeros_hbm)


def _tc_dis(dega):
    def body(dega_ref, dis_ref):
        deg = dega_ref[0, :, 0:1] + dega_ref[1, :, 0:1] + 1.0
        dis_ref[...] = lax.rsqrt(deg)

    return pl.pallas_call(
        body,
        out_shape=jax.ShapeDtypeStruct((NPAD, 1), jnp.float32),
    )(dega)


_BLK = 1280
_GRID = NPAD // _BLK


def _tc_matmul_scale(x, w, dis):
    def body(x_ref, w_ref, dis_ref, g_ref):
        h = jnp.dot(x_ref[...], w_ref[...], preferred_element_type=jnp.float32)
        g_ref[...] = dis_ref[...] * h

    return pl.pallas_call(
        body,
        grid=(_GRID,),
        in_specs=[
            pl.BlockSpec((_BLK, D), lambda i: (i, 0)),
            pl.BlockSpec((D, D), lambda i: (0, 0)),
            pl.BlockSpec((_BLK, 1), lambda i: (i, 0)),
        ],
        out_specs=pl.BlockSpec((_BLK, D), lambda i: (i, 0)),
        out_shape=jax.ShapeDtypeStruct((NPAD, D), jnp.float32),
    )(x, w, dis)


def _tc_combine_matmul(acc, g_prev, dis, w):
    def body(acc_ref, g_ref, dis_ref, w_ref, out_ref):
        pre = dis_ref[...] * (acc_ref[0] + acc_ref[1] + g_ref[...])
        x2 = jnp.maximum(pre, 0.0)
        h = jnp.dot(x2, w_ref[...], preferred_element_type=jnp.float32)
        out_ref[...] = dis_ref[...] * h

    return pl.pallas_call(
        body,
        grid=(_GRID,),
        in_specs=[
            pl.BlockSpec((NC, _BLK, D), lambda i: (0, i, 0)),
            pl.BlockSpec((_BLK, D), lambda i: (i, 0)),
            pl.BlockSpec((_BLK, 1), lambda i: (i, 0)),
            pl.BlockSpec((D, D), lambda i: (0, 0)),
        ],
        out_specs=pl.BlockSpec((_BLK, D), lambda i: (i, 0)),
        out_shape=jax.ShapeDtypeStruct((NPAD, D), jnp.float32),
    )(acc, g_prev, dis, w)


def _tc_final(acc, g_prev, dis):
    def body(acc_ref, g_ref, dis_ref, out_ref):
        out_ref[...] = dis_ref[...] * (acc_ref[0] + acc_ref[1] + g_ref[...])

    return pl.pallas_call(
        body,
        grid=(_GRID,),
        in_specs=[
            pl.BlockSpec((NC, _BLK, D), lambda i: (0, i, 0)),
            pl.BlockSpec((_BLK, D), lambda i: (i, 0)),
            pl.BlockSpec((_BLK, 1), lambda i: (i, 0)),
        ],
        out_specs=pl.BlockSpec((_BLK, D), lambda i: (i, 0)),
        out_shape=jax.ShapeDtypeStruct((NPAD, D), jnp.float32),
    )(acc, g_prev, dis)


def kernel(x, edge_index, W0, W1, W2):
    x_p = jnp.zeros((NPAD, D), jnp.float32).at[:N].set(x)
    src = jnp.full((EPAD,), DUMMY, jnp.int32).at[:E].set(edge_index[0])
    dst = jnp.full((EPAD,), DUMMY, jnp.int32).at[:E].set(edge_index[1])
    src_w = src.reshape(NW, CH, CHUNK)
    dst_w = dst.reshape(NW, CH, CHUNK)
    onesD = jnp.ones((CHUNK, D), jnp.float32)
    zerosD = jnp.zeros((CHUNK, D), jnp.float32)

    dega = _sc_degree(dst_w, onesD, zerosD)
    dis = _tc_dis(dega)

    g0 = _tc_matmul_scale(x_p, W0, dis)
    acc0 = _sc_scatter(g0, src_w, dst_w, zerosD)
    g1 = _tc_combine_matmul(acc0, g0, dis, W1)
    acc1 = _sc_scatter(g1, src_w, dst_w, zerosD)
    g2 = _tc_combine_matmul(acc1, g1, dis, W2)
    acc2 = _sc_scatter(g2, src_w, dst_w, zerosD)

    out_p = _tc_final(acc2, g2, dis)
    return out_p[:N]
